# baseline (speedup 1.0000x reference)
import functools

import jax
import jax.numpy as jnp
from jax import lax
from jax.experimental import pallas as pl
from jax.experimental.pallas import tpu as pltpu
from jax.experimental.pallas import tpu_sc as plsc

B, T, N_EMBED = 4, 2048, 1024
NUM_EXPERTS, TOP_K = 16, 2
TILE = 2048
N_TOKENS = B * T

NUM_WORKERS = 32
CHUNK = N_TOKENS // NUM_WORKERS


def _logits_body(x_ref, w_ref, sw_ref, b_ref, sb_ref, ei_ref, eo_ref,
                 lg_ref, z_ref):
    i = pl.program_id(0)
    ei = ei_ref[:]
    eo = eo_ref[:]
    fei = jnp.sign(ei) * jnp.sqrt(jnp.abs(ei))
    feo = jnp.sign(eo) * jnp.sqrt(jnp.abs(eo))
    nw = w_ref[:] + sw_ref[:] * (feo * fei)
    nb = b_ref[:] + sb_ref[:] * feo
    logits = jax.lax.dot_general(
        nw, x_ref[:], (((1,), (1,)), ((), ())),
        preferred_element_type=jnp.float32) + nb
    lg_ref[:] = logits

    iota = jax.lax.broadcasted_iota(jnp.int32, logits.shape, 0)
    m1 = jnp.max(logits, axis=0, keepdims=True)
    i1 = jnp.min(jnp.where(logits == m1, iota, NUM_EXPERTS), axis=0,
                 keepdims=True)
    masked = jnp.where(iota == i1, -jnp.inf, logits)
    m2 = jnp.max(masked, axis=0, keepdims=True)
    lse = m1 + jnp.log1p(jnp.exp(m2 - m1))
    part = jnp.sum(lse * lse)

    @pl.when(i == 0)
    def _init():
        z_ref[0, 0] = part

    @pl.when(i != 0)
    def _acc():
        z_ref[0, 0] += part


@jax.jit
def _logits_tc(x, w, sw, b2, sb2, ei2, eo2):
    grid = N_TOKENS // TILE
    lgT, zsum = pl.pallas_call(
        _logits_body,
        grid=(grid,),
        in_specs=[
            pl.BlockSpec((TILE, N_EMBED), lambda i: (i, 0)),
            pl.BlockSpec((NUM_EXPERTS, N_EMBED), lambda i: (0, 0)),
            pl.BlockSpec((NUM_EXPERTS, N_EMBED), lambda i: (0, 0)),
            pl.BlockSpec((NUM_EXPERTS, 1), lambda i: (0, 0)),
            pl.BlockSpec((NUM_EXPERTS, 1), lambda i: (0, 0)),
            pl.BlockSpec((1, N_EMBED), lambda i: (0, 0)),
            pl.BlockSpec((NUM_EXPERTS, 1), lambda i: (0, 0)),
        ],
        out_specs=[
            pl.BlockSpec((NUM_EXPERTS, TILE), lambda i: (0, i)),
            pl.BlockSpec(memory_space=pltpu.SMEM),
        ],
        out_shape=[
            jax.ShapeDtypeStruct((NUM_EXPERTS, N_TOKENS), jnp.float32),
            jax.ShapeDtypeStruct((1, 1), jnp.float32),
        ],
    )(x, w, sw, b2, sb2, ei2, eo2)
    return lgT, zsum


@functools.partial(
    pl.kernel,
    out_type=[
        jax.ShapeDtypeStruct((N_TOKENS * NUM_EXPERTS,), jnp.float32),
        jax.ShapeDtypeStruct((N_TOKENS * 2,), jnp.int32),
    ],
    mesh=plsc.VectorSubcoreMesh(core_axis_name="c", subcore_axis_name="s"),
    compiler_params=pltpu.CompilerParams(use_tc_tiling_on_sc=False,
                                         needs_layout_passes=False),
    scratch_types=[
        pltpu.VMEM((NUM_EXPERTS * CHUNK,), jnp.float32),
        pltpu.VMEM((CHUNK * NUM_EXPERTS,), jnp.float32),
        pltpu.VMEM((CHUNK * 2,), jnp.int32),
    ],
)
def _route_sc(lgT_hbm, rout_hbm, idx_hbm, lgT_v, rout_v, idx_v):
    c = lax.axis_index("c")
    s = lax.axis_index("s")
    w = s * 2 + c
    base = w * CHUNK
    for r in range(NUM_EXPERTS):
        pltpu.sync_copy(lgT_hbm.at[r, pl.ds(base, CHUNK)],
                        lgT_v.at[pl.ds(r * CHUNK, CHUNK)])
    iota = lax.iota(jnp.int32, NUM_EXPERTS)
    group = NUM_EXPERTS // 2

    def body(g, carry):
        acc = jnp.zeros((NUM_EXPERTS,), jnp.int32)
        for k in range(group):
            t = g * group + k
            v = plsc.load_gather(
                lgT_v, [iota * CHUNK + t])
            m1 = jnp.max(v)
            i1 = jnp.min(jnp.where(v == m1, iota, NUM_EXPERTS))
            v2 = jnp.where(iota == i1, -jnp.inf, v)
            m2 = jnp.max(v2)
            i2 = jnp.min(jnp.where(v2 == m2, iota, NUM_EXPERTS))
            ev = jnp.exp(jnp.full((NUM_EXPERTS,), m2 - m1, jnp.float32))
            p1v = 1.0 / (1.0 + ev)
            p2v = 1.0 - p1v
            rout_v[pl.ds(t * NUM_EXPERTS, NUM_EXPERTS)] = jnp.where(
                iota == i1, p1v, jnp.where(iota == i2, p2v, 0.0))
            acc = jnp.where(iota == 2 * k, i1, acc)
            acc = jnp.where(iota == 2 * k + 1, i2, acc)
        idx_v[pl.ds(g * NUM_EXPERTS, NUM_EXPERTS)] = acc
        return carry

    lax.fori_loop(0, CHUNK // group, body, 0)
    pltpu.sync_copy(rout_v, rout_hbm.at[pl.ds(base * NUM_EXPERTS,
                                              CHUNK * NUM_EXPERTS)])
    pltpu.sync_copy(idx_v, idx_hbm.at[pl.ds(base * 2, CHUNK * 2)])


def kernel(mh_output, W, sigma_W, b, sigma_b, eps_in, eps_out):
    x = mh_output.reshape(N_TOKENS, N_EMBED)
    b2 = b.reshape(NUM_EXPERTS, 1)
    sb2 = sigma_b.reshape(NUM_EXPERTS, 1)
    ei2 = eps_in.reshape(1, N_EMBED)
    eo2 = eps_out.reshape(NUM_EXPERTS, 1)
    lgT, zsum = _logits_tc(x, W, sigma_W, b2, sb2, ei2, eo2)
    rout, idx = _route_sc(lgT)
    router_output = rout.reshape(B, T, NUM_EXPERTS)
    indices = idx.reshape(B, T, TOP_K)
    z_loss = zsum[0, 0] / jnp.float32(N_TOKENS)
    return router_output, indices, z_loss


# speedup vs baseline: 1.4296x; 1.4296x over previous
import functools

import jax
import jax.numpy as jnp
from jax import lax
from jax.experimental import pallas as pl
from jax.experimental.pallas import tpu as pltpu
from jax.experimental.pallas import tpu_sc as plsc

B, T, N_EMBED = 4, 2048, 1024
NUM_EXPERTS, TOP_K = 16, 2
TILE = 2048
N_TOKENS = B * T

NUM_WORKERS = 32
CHUNK = N_TOKENS // NUM_WORKERS


def _logits_body(x_ref, w_ref, sw_ref, b_ref, sb_ref, ei_ref, eo_ref,
                 lg_ref, z_ref):
    i = pl.program_id(0)
    ei = ei_ref[:]
    eo = eo_ref[:]
    fei = jnp.sign(ei) * jnp.sqrt(jnp.abs(ei))
    feo = jnp.sign(eo) * jnp.sqrt(jnp.abs(eo))
    nw = w_ref[:] + sw_ref[:] * (feo * fei)
    nb = b_ref[:] + sb_ref[:] * feo
    logits = jax.lax.dot_general(
        nw, x_ref[:], (((1,), (1,)), ((), ())),
        preferred_element_type=jnp.float32) + nb
    lg_ref[:] = logits

    iota = jax.lax.broadcasted_iota(jnp.int32, logits.shape, 0)
    m1 = jnp.max(logits, axis=0, keepdims=True)
    i1 = jnp.min(jnp.where(logits == m1, iota, NUM_EXPERTS), axis=0,
                 keepdims=True)
    masked = jnp.where(iota == i1, -jnp.inf, logits)
    m2 = jnp.max(masked, axis=0, keepdims=True)
    lse = m1 + jnp.log1p(jnp.exp(m2 - m1))
    part = jnp.sum(lse * lse)

    @pl.when(i == 0)
    def _init():
        z_ref[0, 0] = part

    @pl.when(i != 0)
    def _acc():
        z_ref[0, 0] += part


@jax.jit
def _logits_tc(x, w, sw, b2, sb2, ei2, eo2):
    grid = N_TOKENS // TILE
    lgT, zsum = pl.pallas_call(
        _logits_body,
        grid=(grid,),
        in_specs=[
            pl.BlockSpec((TILE, N_EMBED), lambda i: (i, 0)),
            pl.BlockSpec((NUM_EXPERTS, N_EMBED), lambda i: (0, 0)),
            pl.BlockSpec((NUM_EXPERTS, N_EMBED), lambda i: (0, 0)),
            pl.BlockSpec((NUM_EXPERTS, 1), lambda i: (0, 0)),
            pl.BlockSpec((NUM_EXPERTS, 1), lambda i: (0, 0)),
            pl.BlockSpec((1, N_EMBED), lambda i: (0, 0)),
            pl.BlockSpec((NUM_EXPERTS, 1), lambda i: (0, 0)),
        ],
        out_specs=[
            pl.BlockSpec((NUM_EXPERTS, TILE), lambda i: (0, i)),
            pl.BlockSpec(memory_space=pltpu.SMEM),
        ],
        out_shape=[
            jax.ShapeDtypeStruct((NUM_EXPERTS, N_TOKENS), jnp.float32),
            jax.ShapeDtypeStruct((1, 1), jnp.float32),
        ],
    )(x, w, sw, b2, sb2, ei2, eo2)
    return lgT, zsum


@functools.partial(
    pl.kernel,
    out_type=[
        jax.ShapeDtypeStruct((N_TOKENS * NUM_EXPERTS,), jnp.float32),
        jax.ShapeDtypeStruct((N_TOKENS * 2,), jnp.int32),
    ],
    mesh=plsc.VectorSubcoreMesh(core_axis_name="c", subcore_axis_name="s"),
    compiler_params=pltpu.CompilerParams(use_tc_tiling_on_sc=False,
                                         needs_layout_passes=False),
    scratch_types=[
        pltpu.VMEM((NUM_EXPERTS * CHUNK,), jnp.float32),
        pltpu.VMEM((CHUNK * NUM_EXPERTS,), jnp.float32),
        pltpu.VMEM((CHUNK * 2,), jnp.int32),
        pltpu.SemaphoreType.DMA,
    ],
)
def _route_sc(lgT_hbm, rout_hbm, idx_hbm, lgT_v, rout_v, idx_v, sem):
    c = lax.axis_index("c")
    s = lax.axis_index("s")
    w = s * 2 + c
    base = w * CHUNK
    copies = [
        pltpu.async_copy(lgT_hbm.at[r, pl.ds(base, CHUNK)],
                         lgT_v.at[pl.ds(r * CHUNK, CHUNK)], sem)
        for r in range(NUM_EXPERTS)
    ]
    for cp in copies:
        cp.wait()
    iota = lax.iota(jnp.int32, NUM_EXPERTS)
    zrow = jnp.zeros((NUM_EXPERTS,), jnp.float32)
    L = NUM_EXPERTS

    def body(g, carry):
        vals = [lgT_v[pl.ds(r * CHUNK + g * L, L)] for r in range(NUM_EXPERTS)]
        nodes = []
        for r in range(NUM_EXPERTS // 2):
            a, b = vals[2 * r], vals[2 * r + 1]
            tb = b > a
            ia = jnp.full((L,), 2 * r, jnp.int32)
            ib = jnp.full((L,), 2 * r + 1, jnp.int32)
            nodes.append((jnp.maximum(a, b), jnp.where(tb, ib, ia),
                          jnp.minimum(a, b), jnp.where(tb, ia, ib)))
        while len(nodes) > 1:
            nxt = []
            for j in range(0, len(nodes), 2):
                m1a, i1a, m2a, i2a = nodes[j]
                m1b, i1b, m2b, i2b = nodes[j + 1]
                tb = m1b > m1a
                m1 = jnp.maximum(m1a, m1b)
                i1 = jnp.where(tb, i1b, i1a)
                lv = jnp.where(tb, m1a, m1b)
                li = jnp.where(tb, i1a, i1b)
                wv = jnp.where(tb, m2b, m2a)
                wi = jnp.where(tb, i2b, i2a)
                tw = (wv > lv) | ((wv == lv) & (wi < li))
                m2 = jnp.where(tw, wv, lv)
                i2 = jnp.where(tw, wi, li)
                nxt.append((m1, i1, m2, i2))
            nodes = nxt
        m1, i1, m2, i2 = nodes[0]

        ev = jnp.exp(m2 - m1)
        p1 = 1.0 / (1.0 + ev)
        p2 = 1.0 - p1
        for k in range(L):
            rout_v[pl.ds((g * L + k) * NUM_EXPERTS, NUM_EXPERTS)] = zrow
        tok = g * L + iota
        plsc.store_scatter(rout_v, [tok * NUM_EXPERTS + i1], p1)
        plsc.store_scatter(rout_v, [tok * NUM_EXPERTS + i2], p2)
        plsc.store_scatter(idx_v, [tok * 2], i1)
        plsc.store_scatter(idx_v, [tok * 2 + 1], i2)
        return carry

    lax.fori_loop(0, CHUNK // L, body, 0)
    pltpu.sync_copy(rout_v, rout_hbm.at[pl.ds(base * NUM_EXPERTS,
                                              CHUNK * NUM_EXPERTS)])
    pltpu.sync_copy(idx_v, idx_hbm.at[pl.ds(base * 2, CHUNK * 2)])


def kernel(mh_output, W, sigma_W, b, sigma_b, eps_in, eps_out):
    x = mh_output.reshape(N_TOKENS, N_EMBED)
    b2 = b.reshape(NUM_EXPERTS, 1)
    sb2 = sigma_b.reshape(NUM_EXPERTS, 1)
    ei2 = eps_in.reshape(1, N_EMBED)
    eo2 = eps_out.reshape(NUM_EXPERTS, 1)
    lgT, zsum = _logits_tc(x, W, sigma_W, b2, sb2, ei2, eo2)
    rout, idx = _route_sc(lgT)
    router_output = rout.reshape(B, T, NUM_EXPERTS)
    indices = idx.reshape(B, T, TOP_K)
    z_loss = zsum[0, 0] / jnp.float32(N_TOKENS)
    return router_output, indices, z_loss


# speedup vs baseline: 3.4082x; 2.3840x over previous
import functools

import jax
import jax.numpy as jnp
from jax.experimental import pallas as pl
from jax.experimental.pallas import tpu as pltpu

B, T, N_EMBED = 4, 2048, 1024
NUM_EXPERTS, TOP_K = 16, 2
TILE = 2048
N_TOKENS = B * T


def _router_body(x_ref, w_ref, sw_ref, b_ref, sb_ref, ei_ref, eo_ref,
                 rout_ref, idx_ref, z_ref):
    i = pl.program_id(0)
    ei = ei_ref[:]
    eo = eo_ref[:]
    fei = jnp.sign(ei) * jnp.sqrt(jnp.abs(ei))
    feo = jnp.sign(eo) * jnp.sqrt(jnp.abs(eo))
    nw = w_ref[:] + sw_ref[:] * (feo * fei)
    nb = b_ref[:] + sb_ref[:] * feo
    logits = jax.lax.dot_general(
        nw, x_ref[:], (((1,), (1,)), ((), ())),
        preferred_element_type=jnp.float32) + nb

    iota = jax.lax.broadcasted_iota(jnp.int32, logits.shape, 0)
    m1 = jnp.max(logits, axis=0, keepdims=True)
    i1 = jnp.min(jnp.where(logits == m1, iota, NUM_EXPERTS), axis=0,
                 keepdims=True)
    sel1 = iota == i1
    masked = jnp.where(sel1, -jnp.inf, logits)
    m2 = jnp.max(masked, axis=0, keepdims=True)
    i2 = jnp.min(jnp.where(masked == m2, iota, NUM_EXPERTS), axis=0,
                 keepdims=True)
    sel2 = iota == i2

    e = jnp.exp(m2 - m1)
    denom = 1.0 + e
    p1 = 1.0 / denom
    p2 = e / denom
    rout_ref[:] = jnp.where(sel1, p1, jnp.where(sel2, p2, 0.0))
    idx_ref[0:1, :] = i1
    idx_ref[1:2, :] = i2

    lse = m1 + jnp.log1p(e)
    part = jnp.sum(lse * lse)

    @pl.when(i == 0)
    def _init():
        z_ref[0, 0] = part

    @pl.when(i != 0)
    def _acc():
        z_ref[0, 0] += part


@jax.jit
def _router(x, w, sw, b2, sb2, ei2, eo2):
    grid = N_TOKENS // TILE
    rout, idx, zsum = pl.pallas_call(
        _router_body,
        grid=(grid,),
        in_specs=[
            pl.BlockSpec((TILE, N_EMBED), lambda i: (i, 0)),
            pl.BlockSpec((NUM_EXPERTS, N_EMBED), lambda i: (0, 0)),
            pl.BlockSpec((NUM_EXPERTS, N_EMBED), lambda i: (0, 0)),
            pl.BlockSpec((NUM_EXPERTS, 1), lambda i: (0, 0)),
            pl.BlockSpec((NUM_EXPERTS, 1), lambda i: (0, 0)),
            pl.BlockSpec((1, N_EMBED), lambda i: (0, 0)),
            pl.BlockSpec((NUM_EXPERTS, 1), lambda i: (0, 0)),
        ],
        out_specs=[
            pl.BlockSpec((NUM_EXPERTS, TILE), lambda i: (0, i)),
            pl.BlockSpec((2, TILE), lambda i: (0, i)),
            pl.BlockSpec(memory_space=pltpu.SMEM),
        ],
        out_shape=[
            jax.ShapeDtypeStruct((NUM_EXPERTS, N_TOKENS), jnp.float32),
            jax.ShapeDtypeStruct((2, N_TOKENS), jnp.int32),
            jax.ShapeDtypeStruct((1, 1), jnp.float32),
        ],
    )(x, w, sw, b2, sb2, ei2, eo2)
    return rout, idx, zsum


def kernel(mh_output, W, sigma_W, b, sigma_b, eps_in, eps_out):
    x = mh_output.reshape(N_TOKENS, N_EMBED)
    b2 = b.reshape(NUM_EXPERTS, 1)
    sb2 = sigma_b.reshape(NUM_EXPERTS, 1)
    ei2 = eps_in.reshape(1, N_EMBED)
    eo2 = eps_out.reshape(NUM_EXPERTS, 1)
    rout, idx, zsum = _router(x, W, sigma_W, b2, sb2, ei2, eo2)
    router_output = rout.T.reshape(B, T, NUM_EXPERTS)
    indices = idx.T.reshape(B, T, TOP_K)
    z_loss = zsum[0, 0] / jnp.float32(N_TOKENS)
    return router_output, indices, z_loss
